# baseline (speedup 1.0000x reference)
import functools

import jax
import jax.numpy as jnp
import numpy as np
from jax.experimental import pallas as pl
from jax.experimental.pallas import tpu as pltpu


def _flat_w(w):
    O, C = w.shape[0], w.shape[1]
    return jnp.transpose(w, (0, 2, 3, 1)).reshape(O, 9 * C)


def _boundary_mask(H, W, C):
    yy = np.arange(H).reshape(H, 1)
    xx = np.arange(W).reshape(1, W)
    rows = []
    for ky in range(3):
        for kx in range(3):
            dy, dx = ky - 1, kx - 1
            ok = ((yy + dy >= 0) & (yy + dy < H) &
                  (xx + dx >= 0) & (xx + dx < W)).reshape(1, H * W)
            rows.append(np.broadcast_to(ok, (C, H * W)))
    return np.concatenate(rows, axis=0).astype(np.float32)


def _grucell_kernel(x_ref, h0_ref, wx_ref, whg_ref, whc_ref, bg_ref, bc_ref,
                    mask_ref, out_ref, hout_ref, h_s,
                    *, n_grid, ts, nf, Cin, H, W, B):
    HW = H * W
    offs = [(ky - 1) * W + (kx - 1) for ky in range(3) for kx in range(3)]
    mask = mask_ref[...]
    g = pl.program_id(0)

    C4 = mask.shape[0] * 4 // 9
    m_lo, m_hi = mask[:C4], mask[-C4:]

    def window(v):
        L = v.shape[1]
        taps = [v if o == 0 else pltpu.roll(v, (-o) % L, axis=1)
                for o in offs]
        return jnp.concatenate(
            [jnp.concatenate(taps[:4], axis=0) * m_lo, taps[4],
             jnp.concatenate(taps[5:], axis=0) * m_hi], axis=0)

    @pl.when(g == 0)
    def _init():
        h_s[...] = jnp.concatenate([h0_ref[i] for i in range(B)], axis=1)

    h = h_s[...]
    for s in range(ts):
        winx = window(jnp.concatenate(
            [x_ref[s, i].astype(jnp.bfloat16) for i in range(B)], axis=1))
        xa = jnp.dot(wx_ref[...], winx,
                     preferred_element_type=jnp.float32)
        hb = h.astype(jnp.bfloat16)
        winh = window(hb)
        gates = jax.nn.sigmoid(
            jnp.dot(whg_ref[...], winh, preferred_element_type=jnp.float32)
            + xa[:2 * nf] + bg_ref[...])
        reset, update = gates[:nf], gates[nf:]
        rhb = (reset * h).astype(jnp.bfloat16)
        winrh = window(rhb)
        cand = jnp.tanh(
            jnp.dot(whc_ref[...], winrh, preferred_element_type=jnp.float32)
            + xa[2 * nf:] + bc_ref[...])
        h = h + update * (cand - h)
        hb16 = h.astype(jnp.bfloat16)
        for i in range(B):
            out_ref[s, i] = hb16[:, i * HW:(i + 1) * HW]
    h_s[...] = h

    @pl.when(g == n_grid - 1)
    def _fin():
        for i in range(B):
            hout_ref[i] = h[:, i * HW:(i + 1) * HW]


def kernel(inputs, states, wg, bg, wc, bc):
    seq_len, B, Cin, H, W = inputs.shape
    nf = wc.shape[0]
    HW = H * W
    assert Cin == nf, "shared boundary mask assumes Cin == nf"
    ts = 2 if seq_len % 2 == 0 else 1
    n_grid = seq_len // ts

    x = inputs.reshape(seq_len, B, Cin, HW)
    h0 = (jnp.zeros((B, nf, HW), jnp.float32) if states is None
          else states.reshape(B, nf, HW).astype(jnp.float32))

    wg_f = wg.astype(jnp.float32)
    wc_f = wc.astype(jnp.float32)
    wx = jnp.concatenate([_flat_w(wg_f[:, :Cin]), _flat_w(wc_f[:, :Cin])],
                         axis=0).astype(jnp.bfloat16)
    whg = _flat_w(wg_f[:, Cin:]).astype(jnp.bfloat16)
    whc = _flat_w(wc_f[:, Cin:]).astype(jnp.bfloat16)
    bgv = bg.astype(jnp.float32).reshape(2 * nf, 1)
    bcv = bc.astype(jnp.float32).reshape(nf, 1)
    mask = jnp.asarray(np.tile(_boundary_mask(H, W, nf), (1, B)),
                       dtype=jnp.bfloat16)

    kfn = functools.partial(_grucell_kernel, n_grid=n_grid, ts=ts, nf=nf,
                            Cin=Cin, H=H, W=W, B=B)

    def const(shape):
        return pl.BlockSpec(shape, lambda t: (0,) * len(shape))

    out_seq, h_final = pl.pallas_call(
        kfn,
        grid=(n_grid,),
        in_specs=[
            pl.BlockSpec((ts, B, Cin, HW), lambda t: (t, 0, 0, 0)),
            const((B, nf, HW)),
            const(wx.shape),
            const(whg.shape),
            const(whc.shape),
            const(bgv.shape),
            const(bcv.shape),
            const(mask.shape),
        ],
        out_specs=[
            pl.BlockSpec((ts, B, nf, HW), lambda t: (t, 0, 0, 0)),
            const((B, nf, HW)),
        ],
        out_shape=(
            jax.ShapeDtypeStruct((seq_len, B, nf, HW), jnp.bfloat16),
            jax.ShapeDtypeStruct((B, nf, HW), jnp.float32),
        ),
        scratch_shapes=[
            pltpu.VMEM((nf, B * HW), jnp.float32),
        ],
        compiler_params=pltpu.CompilerParams(
            dimension_semantics=("arbitrary",),
            vmem_limit_bytes=64 * 1024 * 1024),
    )(x, h0, wx, whg, whc, bgv, bcv, mask)

    outputs = out_seq.reshape(seq_len, B, nf, H, W).astype(jnp.float32)
    return outputs, h_final.reshape(B, nf, H, W)


# speedup vs baseline: 1.1406x; 1.1406x over previous
import functools

import jax
import jax.numpy as jnp
import numpy as np
from jax.experimental import pallas as pl
from jax.experimental.pallas import tpu as pltpu


def _flat_w(w):
    O, C = w.shape[0], w.shape[1]
    return jnp.transpose(w, (0, 2, 3, 1)).reshape(O, 9 * C)


def _boundary_mask(H, W, C):
    yy = np.arange(H).reshape(H, 1)
    xx = np.arange(W).reshape(1, W)
    rows = []
    for ky in range(3):
        for kx in range(3):
            dy, dx = ky - 1, kx - 1
            ok = ((yy + dy >= 0) & (yy + dy < H) &
                  (xx + dx >= 0) & (xx + dx < W)).reshape(1, H * W)
            rows.append(np.broadcast_to(ok, (C, H * W)))
    return np.concatenate(rows, axis=0).astype(np.float32)


def _grucell_kernel(x_ref, h0_ref, wx_ref, wh_ref, b_ref,
                    mask_ref, out_ref, hout_ref, h_s,
                    *, n_grid, ts, nf, Cin, H, W, B):
    HW = H * W
    offs = [(ky - 1) * W + (kx - 1) for ky in range(3) for kx in range(3)]
    mask = mask_ref[...]
    g = pl.program_id(0)

    C4 = mask.shape[0] * 4 // 9
    m_lo, m_hi = mask[:C4], mask[-C4:]

    def window(v):
        taps = [v if o == 0 else pltpu.roll(v, (-o) % HW, axis=1)
                for o in offs]
        return jnp.concatenate(
            [jnp.concatenate(taps[:4], axis=0) * m_lo, taps[4],
             jnp.concatenate(taps[5:], axis=0) * m_hi], axis=0)

    def batch_window(img):
        return jnp.concatenate([window(img(i)) for i in range(B)], axis=1)

    @pl.when(g == 0)
    def _init():
        h_s[...] = jnp.concatenate([h0_ref[i] for i in range(B)], axis=1)

    h = h_s[...]
    for s in range(ts):
        winx = batch_window(lambda i: x_ref[s, i].astype(jnp.bfloat16))
        xa = jnp.dot(wx_ref[...], winx,
                     preferred_element_type=jnp.float32)
        hb = h.astype(jnp.bfloat16)
        winh = batch_window(lambda i: hb[:, i * HW:(i + 1) * HW])
        gates = jax.nn.sigmoid(
            jnp.dot(wh_ref[:2 * nf], winh,
                    preferred_element_type=jnp.float32)
            + xa[:2 * nf] + b_ref[:2 * nf])
        reset, update = gates[:nf], gates[nf:]
        rhb = (reset * h).astype(jnp.bfloat16)
        winrh = batch_window(lambda i: rhb[:, i * HW:(i + 1) * HW])
        cand = jnp.tanh(
            jnp.dot(wh_ref[2 * nf:], winrh,
                    preferred_element_type=jnp.float32)
            + xa[2 * nf:] + b_ref[2 * nf:])
        h = h + update * (cand - h)
        hb16 = h.astype(jnp.bfloat16)
        for i in range(B):
            out_ref[s, i] = hb16[:, i * HW:(i + 1) * HW]
    h_s[...] = h

    @pl.when(g == n_grid - 1)
    def _fin():
        for i in range(B):
            hout_ref[i] = h[:, i * HW:(i + 1) * HW]


def kernel(inputs, states, wg, bg, wc, bc):
    seq_len, B, Cin, H, W = inputs.shape
    nf = wc.shape[0]
    HW = H * W
    assert Cin == nf, "shared boundary mask assumes Cin == nf"
    ts = 2 if seq_len % 2 == 0 else 1
    n_grid = seq_len // ts

    x = inputs.reshape(seq_len, B, Cin, HW)
    h0 = (jnp.zeros((B, nf, HW), jnp.float32) if states is None
          else states.reshape(B, nf, HW).astype(jnp.float32))

    wall = jnp.concatenate([wg, wc], axis=0).astype(jnp.float32)
    wflat = jnp.transpose(wall, (0, 2, 3, 1)).astype(jnp.bfloat16)
    wx = wflat[:, :, :, :Cin].reshape(3 * nf, 9 * Cin)
    wh = wflat[:, :, :, Cin:].reshape(3 * nf, 9 * nf)
    bv = jnp.concatenate([bg, bc]).astype(jnp.float32).reshape(3 * nf, 1)
    mask = jnp.asarray(_boundary_mask(H, W, nf), dtype=jnp.bfloat16)

    kfn = functools.partial(_grucell_kernel, n_grid=n_grid, ts=ts, nf=nf,
                            Cin=Cin, H=H, W=W, B=B)

    def const(shape):
        return pl.BlockSpec(shape, lambda t: (0,) * len(shape))

    out_seq, h_final = pl.pallas_call(
        kfn,
        grid=(n_grid,),
        in_specs=[
            pl.BlockSpec((ts, B, Cin, HW), lambda t: (t, 0, 0, 0)),
            const((B, nf, HW)),
            const(wx.shape),
            const(wh.shape),
            const(bv.shape),
            const(mask.shape),
        ],
        out_specs=[
            pl.BlockSpec((ts, B, nf, HW), lambda t: (t, 0, 0, 0)),
            const((B, nf, HW)),
        ],
        out_shape=(
            jax.ShapeDtypeStruct((seq_len, B, nf, HW), jnp.bfloat16),
            jax.ShapeDtypeStruct((B, nf, HW), jnp.float32),
        ),
        scratch_shapes=[
            pltpu.VMEM((nf, B * HW), jnp.float32),
        ],
        compiler_params=pltpu.CompilerParams(
            dimension_semantics=("arbitrary",),
            vmem_limit_bytes=64 * 1024 * 1024),
    )(x, h0, wx, wh, bv, mask)

    outputs = out_seq.reshape(seq_len, B, nf, H, W).astype(jnp.float32)
    return outputs, h_final.reshape(B, nf, H, W)


# speedup vs baseline: 1.1511x; 1.0092x over previous
import functools

import jax
import jax.numpy as jnp
import numpy as np
from jax.experimental import pallas as pl
from jax.experimental.pallas import tpu as pltpu


def _flat_w(w):
    O, C = w.shape[0], w.shape[1]
    return jnp.transpose(w, (0, 2, 3, 1)).reshape(O, 9 * C)


def _boundary_mask(H, W, C):
    yy = np.arange(H).reshape(H, 1)
    xx = np.arange(W).reshape(1, W)
    rows = []
    for ky in range(3):
        for kx in range(3):
            dy, dx = ky - 1, kx - 1
            ok = ((yy + dy >= 0) & (yy + dy < H) &
                  (xx + dx >= 0) & (xx + dx < W)).reshape(1, H * W)
            rows.append(np.broadcast_to(ok, (C, H * W)))
    return np.concatenate(rows, axis=0).astype(np.float32)


def _grucell_kernel(x_ref, h0_ref, wx_ref, wh_ref, b_ref,
                    mask_ref, out_ref, hout_ref, h_s,
                    *, n_grid, ts, nf, Cin, H, W, B):
    HW = H * W
    offs = [(ky - 1) * W + (kx - 1) for ky in range(3) for kx in range(3)]
    mask = mask_ref[...]
    g = pl.program_id(0)

    C4 = mask.shape[0] * 4 // 9
    m_lo, m_hi = mask[:C4], mask[-C4:]

    def window(v):
        taps = [v if o == 0 else pltpu.roll(v, (-o) % HW, axis=1)
                for o in offs]
        return jnp.concatenate(
            [jnp.concatenate(taps[:4], axis=0) * m_lo, taps[4],
             jnp.concatenate(taps[5:], axis=0) * m_hi], axis=0)

    def batch_window(img):
        return jnp.concatenate([window(img(i)) for i in range(B)], axis=1)

    @pl.when(g == 0)
    def _init():
        h_s[...] = jnp.concatenate([h0_ref[i] for i in range(B)], axis=1)

    h = h_s[...]
    for s in range(ts):
        winx = batch_window(lambda i: x_ref[s, i].astype(jnp.bfloat16))
        xa = jnp.dot(wx_ref[...], winx,
                     preferred_element_type=jnp.float32)
        hb = h.astype(jnp.bfloat16)
        winh = batch_window(lambda i: hb[:, i * HW:(i + 1) * HW])
        gates = jax.nn.sigmoid(
            jnp.dot(wh_ref[:2 * nf], winh,
                    preferred_element_type=jnp.float32)
            + xa[:2 * nf] + b_ref[:2 * nf])
        reset, update = gates[:nf], gates[nf:]
        rhb = (reset * h).astype(jnp.bfloat16)
        winrh = batch_window(lambda i: rhb[:, i * HW:(i + 1) * HW])
        cand = jnp.tanh(
            jnp.dot(wh_ref[2 * nf:], winrh,
                    preferred_element_type=jnp.float32)
            + xa[2 * nf:] + b_ref[2 * nf:])
        h = h + update * (cand - h)
        hb16 = h.astype(jnp.bfloat16)
        for i in range(B):
            out_ref[s, i] = hb16[:, i * HW:(i + 1) * HW]
    h_s[...] = h

    @pl.when(g == n_grid - 1)
    def _fin():
        for i in range(B):
            hout_ref[i] = h[:, i * HW:(i + 1) * HW]


def kernel(inputs, states, wg, bg, wc, bc):
    seq_len, B, Cin, H, W = inputs.shape
    nf = wc.shape[0]
    HW = H * W
    assert Cin == nf, "shared boundary mask assumes Cin == nf"
    ts = 4 if seq_len % 4 == 0 else (2 if seq_len % 2 == 0 else 1)
    n_grid = seq_len // ts

    x = inputs.reshape(seq_len, B, Cin, HW)
    h0 = (jnp.zeros((B, nf, HW), jnp.float32) if states is None
          else states.reshape(B, nf, HW).astype(jnp.float32))

    wall = jnp.concatenate([wg, wc], axis=0).astype(jnp.float32)
    wflat = jnp.transpose(wall, (0, 2, 3, 1)).astype(jnp.bfloat16)
    wx = wflat[:, :, :, :Cin].reshape(3 * nf, 9 * Cin)
    wh = wflat[:, :, :, Cin:].reshape(3 * nf, 9 * nf)
    bv = jnp.concatenate([bg, bc]).astype(jnp.float32).reshape(3 * nf, 1)
    mask = jnp.asarray(_boundary_mask(H, W, nf), dtype=jnp.bfloat16)

    kfn = functools.partial(_grucell_kernel, n_grid=n_grid, ts=ts, nf=nf,
                            Cin=Cin, H=H, W=W, B=B)

    def const(shape):
        return pl.BlockSpec(shape, lambda t: (0,) * len(shape))

    out_seq, h_final = pl.pallas_call(
        kfn,
        grid=(n_grid,),
        in_specs=[
            pl.BlockSpec((ts, B, Cin, HW), lambda t: (t, 0, 0, 0)),
            const((B, nf, HW)),
            const(wx.shape),
            const(wh.shape),
            const(bv.shape),
            const(mask.shape),
        ],
        out_specs=[
            pl.BlockSpec((ts, B, nf, HW), lambda t: (t, 0, 0, 0)),
            const((B, nf, HW)),
        ],
        out_shape=(
            jax.ShapeDtypeStruct((seq_len, B, nf, HW), jnp.bfloat16),
            jax.ShapeDtypeStruct((B, nf, HW), jnp.float32),
        ),
        scratch_shapes=[
            pltpu.VMEM((nf, B * HW), jnp.float32),
        ],
        compiler_params=pltpu.CompilerParams(
            dimension_semantics=("arbitrary",),
            vmem_limit_bytes=64 * 1024 * 1024),
    )(x, h0, wx, wh, bv, mask)

    outputs = out_seq.reshape(seq_len, B, nf, H, W).astype(jnp.float32)
    return outputs, h_final.reshape(B, nf, H, W)
